# baseline (speedup 1.0000x reference)
import jax
import jax.numpy as jnp
from jax import lax
from jax.experimental import pallas as pl
from jax.experimental.pallas import tpu as pltpu
from jax.experimental.pallas import tpu_sc as plsc

N = 10000
D = 128
E = 320000

NC = 2
NS = 16
NW = NC * NS

EPT = 10240
EPAD = EPT * NW
NPADROWS = 112
NACC = N + NPADROWS
RPT = NACC // NS

CH = 64
NCH = EPT // CH
RB = 4
IB = 8
NIB = NCH // IB

CHC = 128
NCHC = EPT // CHC
IBC = 8
NIBC = NCHC // IBC

_DOT = dict(preferred_element_type=jnp.float32, precision=lax.Precision.HIGHEST)


def _xwt(a, w):
    return lax.dot_general(a, w, (((1,), (1,)), ((), ())), **_DOT)


def _sc_body(table, srcg, dstg, zacc, agg_out,
             src_v, dst_v, r0, r1, r2, r3, acc_sh, s0, s1, s2, s3):
    c = lax.axis_index("c")
    s = lax.axis_index("s")
    wid = c * NS + s
    lo = s * RPT

    pltpu.sync_copy(zacc.at[pl.ds(lo, RPT)], acc_sh.at[pl.ds(lo, RPT)])
    plsc.subcore_barrier()

    bufs = (r0, r1, r2, r3)
    sems = (s0, s1, s2, s3)

    def outer(jb, carry):
        ib0 = wid * NCH + jb * IB
        pltpu.sync_copy(srcg.at[pl.ds(ib0, IB)], src_v)
        pltpu.sync_copy(dstg.at[pl.ds(ib0, IB)], dst_v)

        h = [None] * IB
        for j in range(RB - 1):
            h[j] = pltpu.async_copy(table.at[src_v.at[j]], bufs[j % RB],
                                    sems[j % RB])
        for j in range(IB):
            jn = j + RB - 1
            if jn < IB:
                h[jn] = pltpu.async_copy(table.at[src_v.at[jn]],
                                         bufs[jn % RB], sems[jn % RB])
            h[j].wait()
            pltpu.sync_copy(bufs[j % RB], acc_sh.at[dst_v.at[j]], add=True)
        return carry

    lax.fori_loop(0, NIB, outer, 0)
    plsc.subcore_barrier()

    dlo = c * NACC + lo
    pltpu.sync_copy(acc_sh.at[pl.ds(lo, RPT)], agg_out.at[pl.ds(dlo, RPT)])


_sc_seg = pl.kernel(
    _sc_body,
    out_type=jax.ShapeDtypeStruct((NC * NACC, D), jnp.float32),
    mesh=plsc.VectorSubcoreMesh(core_axis_name="c", subcore_axis_name="s"),
    scratch_types=[
        pltpu.VMEM((IB, CH), jnp.int32),
        pltpu.VMEM((IB, CH), jnp.int32),
        pltpu.VMEM((CH, D), jnp.float32),
        pltpu.VMEM((CH, D), jnp.float32),
        pltpu.VMEM((CH, D), jnp.float32),
        pltpu.VMEM((CH, D), jnp.float32),
        pltpu.VMEM_SHARED((NACC, D), jnp.float32),
        pltpu.SemaphoreType.DMA,
        pltpu.SemaphoreType.DMA,
        pltpu.SemaphoreType.DMA,
        pltpu.SemaphoreType.DMA,
    ],
)


def _cnt_body(dstg, zcnt, ones, cnt_out, dst_v, ones_v, cnt_sh):
    c = lax.axis_index("c")
    s = lax.axis_index("s")
    wid = c * NS + s
    lo = s * RPT

    pltpu.sync_copy(zcnt.at[pl.ds(lo, RPT)], cnt_sh.at[pl.ds(lo, RPT)])
    pltpu.sync_copy(ones, ones_v)
    plsc.subcore_barrier()

    def outer(jb, carry):
        ib0 = wid * NCHC + jb * IBC
        pltpu.sync_copy(dstg.at[pl.ds(ib0, IBC)], dst_v)

        def inner(j, c2):
            pltpu.sync_copy(ones_v, cnt_sh.at[dst_v.at[j]], add=True)
            return c2

        return lax.fori_loop(0, IBC, inner, carry)

    lax.fori_loop(0, NIBC, outer, 0)
    plsc.subcore_barrier()

    dlo = c * NACC + lo
    pltpu.sync_copy(cnt_sh.at[pl.ds(lo, RPT)], cnt_out.at[pl.ds(dlo, RPT)])


_sc_cnt = pl.kernel(
    _cnt_body,
    out_type=jax.ShapeDtypeStruct((NC * NACC, D), jnp.float32),
    mesh=plsc.VectorSubcoreMesh(core_axis_name="c", subcore_axis_name="s"),
    scratch_types=[
        pltpu.VMEM((IBC, CHC), jnp.int32),
        pltpu.VMEM((CHC, D), jnp.float32),
        pltpu.VMEM_SHARED((NACC, D), jnp.float32),
    ],
)


_BR = 1000


def _mid_body(a0, a1, c0, c1, x_ref, wl, wr, b, h_ref):
    cnt = c0[...] + c1[...]
    inv = 1.0 / jnp.maximum(cnt, 1.0)
    mean = (a0[...] + a1[...]) * inv
    h = _xwt(mean, wl[...]) + b[...] + _xwt(x_ref[...], wr[...])
    h_ref[...] = jnp.maximum(h, 0.0)


def _mid(a0v, a1v, c0v, c1v, x, wl, wr, b):
    blk = pl.BlockSpec((_BR, D), lambda i: (i, 0))
    cblk = pl.BlockSpec((_BR, 1), lambda i: (i, 0))
    wblk = pl.BlockSpec((D, D), lambda i: (0, 0))
    bblk = pl.BlockSpec((1, D), lambda i: (0, 0))
    return pl.pallas_call(
        _mid_body,
        grid=(N // _BR,),
        in_specs=[blk, blk, cblk, cblk, blk, wblk, wblk, bblk],
        out_specs=blk,
        out_shape=jax.ShapeDtypeStruct((N, D), jnp.float32),
    )(a0v, a1v, c0v, c1v, x, wl, wr, b)


def _fin_body(a0, a1, c0, c1, h_ref, wl, wr, b, o_ref):
    cnt = c0[...] + c1[...]
    inv = 1.0 / jnp.maximum(cnt, 1.0)
    mean = (a0[...] + a1[...]) * inv
    o_ref[...] = _xwt(mean, wl[...]) + b[...] + _xwt(h_ref[...], wr[...])


def _fin(a0v, a1v, c0v, c1v, h, wl, wr, b):
    blk = pl.BlockSpec((_BR, D), lambda i: (i, 0))
    cblk = pl.BlockSpec((_BR, 1), lambda i: (i, 0))
    wblk = pl.BlockSpec((D, D), lambda i: (0, 0))
    bblk = pl.BlockSpec((1, D), lambda i: (0, 0))
    return pl.pallas_call(
        _fin_body,
        grid=(N // _BR,),
        in_specs=[blk, blk, cblk, cblk, blk, wblk, wblk, bblk],
        out_specs=blk,
        out_shape=jax.ShapeDtypeStruct((N, D), jnp.float32),
    )(a0v, a1v, c0v, c1v, h, wl, wr, b)


def kernel(x, edge_index, Wl1, bl1, Wr1, Wl2, bl2, Wr2):
    src = edge_index[0].astype(jnp.int32)
    dst = edge_index[1].astype(jnp.int32)
    padn = EPAD - E
    ar = jnp.arange(padn, dtype=jnp.int32)
    pad_src = (ar * 37) % N
    pad_dst = N + (ar % NPADROWS)
    srcp = jnp.concatenate([src, pad_src])
    dstp = jnp.concatenate([dst, pad_dst])
    srcg = srcp.reshape(NW * NCH, CH)
    dstg = dstp.reshape(NW * NCH, CH)
    dstgc = dstp.reshape(NW * NCHC, CHC)
    zacc = jnp.zeros((NACC, D), jnp.float32)
    ones = jnp.ones((CHC, D), jnp.float32)
    bl1r = bl1.reshape(1, D)
    bl2r = bl2.reshape(1, D)

    cntf = _sc_cnt(dstgc, zacc, ones)
    cntc = cntf[:, :1]
    c0, c1 = cntc[:N], cntc[NACC:NACC + N]

    aggf = _sc_seg(x, srcg, dstg, zacc)
    h = _mid(aggf[:N], aggf[NACC:NACC + N], c0, c1, x, Wl1, Wr1, bl1r)
    aggf2 = _sc_seg(h, srcg, dstg, zacc)
    return _fin(aggf2[:N], aggf2[NACC:NACC + N], c0, c1, h, Wl2, Wr2, bl2r)


# speedup vs baseline: 1.1467x; 1.1467x over previous
import jax
import jax.numpy as jnp
from jax import lax
from jax.experimental import pallas as pl
from jax.experimental.pallas import tpu as pltpu
from jax.experimental.pallas import tpu_sc as plsc

N = 10000
D = 128
E = 320000

NC = 2
NS = 16
NW = NC * NS

EPT = 10240
EPAD = EPT * NW
NPADROWS = 112
NACC = N + NPADROWS
RPT = NACC // NS

CH = 128
NCH = EPT // CH
RB = 2
IB = 16
NIB = NCH // IB

CHC = 128
NCHC = EPT // CHC
IBC = 8
NIBC = NCHC // IBC

_DOT = dict(preferred_element_type=jnp.float32, precision=lax.Precision.HIGHEST)


def _xwt(a, w):
    return lax.dot_general(a, w, (((1,), (1,)), ((), ())), **_DOT)


def _sc_body(table, srcg, dstg, zacc, agg_out,
             src_v, dst_v, r0, r1, acc_sh, s0, s1):
    c = lax.axis_index("c")
    s = lax.axis_index("s")
    wid = c * NS + s
    lo = s * RPT

    pltpu.sync_copy(zacc.at[pl.ds(lo, RPT)], acc_sh.at[pl.ds(lo, RPT)])
    plsc.subcore_barrier()

    bufs = (r0, r1)
    sems = (s0, s1)

    def outer(jb, carry):
        ib0 = wid * NCH + jb * IB
        pltpu.sync_copy(srcg.at[pl.ds(ib0, IB)], src_v)
        pltpu.sync_copy(dstg.at[pl.ds(ib0, IB)], dst_v)

        h = [None] * IB
        for j in range(RB - 1):
            h[j] = pltpu.async_copy(table.at[src_v.at[j]], bufs[j % RB],
                                    sems[j % RB])
        for j in range(IB):
            jn = j + RB - 1
            if jn < IB:
                h[jn] = pltpu.async_copy(table.at[src_v.at[jn]],
                                         bufs[jn % RB], sems[jn % RB])
            h[j].wait()
            pltpu.sync_copy(bufs[j % RB], acc_sh.at[dst_v.at[j]], add=True)
        return carry

    lax.fori_loop(0, NIB, outer, 0)
    plsc.subcore_barrier()

    dlo = c * NACC + lo
    pltpu.sync_copy(acc_sh.at[pl.ds(lo, RPT)], agg_out.at[pl.ds(dlo, RPT)])


_sc_seg = pl.kernel(
    _sc_body,
    out_type=jax.ShapeDtypeStruct((NC * NACC, D), jnp.float32),
    mesh=plsc.VectorSubcoreMesh(core_axis_name="c", subcore_axis_name="s"),
    scratch_types=[
        pltpu.VMEM((IB, CH), jnp.int32),
        pltpu.VMEM((IB, CH), jnp.int32),
        pltpu.VMEM((CH, D), jnp.float32),
        pltpu.VMEM((CH, D), jnp.float32),
        pltpu.VMEM_SHARED((NACC, D), jnp.float32),
        pltpu.SemaphoreType.DMA,
        pltpu.SemaphoreType.DMA,
    ],
)


def _cnt_body(dstg, zcnt, ones, cnt_out, dst_v, ones_v, cnt_sh):
    c = lax.axis_index("c")
    s = lax.axis_index("s")
    wid = c * NS + s
    lo = s * RPT

    pltpu.sync_copy(zcnt.at[pl.ds(lo, RPT)], cnt_sh.at[pl.ds(lo, RPT)])
    pltpu.sync_copy(ones, ones_v)
    plsc.subcore_barrier()

    def outer(jb, carry):
        ib0 = wid * NCHC + jb * IBC
        pltpu.sync_copy(dstg.at[pl.ds(ib0, IBC)], dst_v)

        def inner(j, c2):
            pltpu.sync_copy(ones_v, cnt_sh.at[dst_v.at[j]], add=True)
            return c2

        return lax.fori_loop(0, IBC, inner, carry)

    lax.fori_loop(0, NIBC, outer, 0)
    plsc.subcore_barrier()

    dlo = c * NACC + lo
    pltpu.sync_copy(cnt_sh.at[pl.ds(lo, RPT)], cnt_out.at[pl.ds(dlo, RPT)])


_sc_cnt = pl.kernel(
    _cnt_body,
    out_type=jax.ShapeDtypeStruct((NC * NACC, D), jnp.float32),
    mesh=plsc.VectorSubcoreMesh(core_axis_name="c", subcore_axis_name="s"),
    scratch_types=[
        pltpu.VMEM((IBC, CHC), jnp.int32),
        pltpu.VMEM((CHC, D), jnp.float32),
        pltpu.VMEM_SHARED((NACC, D), jnp.float32),
    ],
)


_BR = 1000


def _mid_body(a0, a1, c0, c1, x_ref, wl, wr, b, h_ref):
    cnt = c0[...] + c1[...]
    inv = 1.0 / jnp.maximum(cnt, 1.0)
    mean = (a0[...] + a1[...]) * inv
    h = _xwt(mean, wl[...]) + b[...] + _xwt(x_ref[...], wr[...])
    h_ref[...] = jnp.maximum(h, 0.0)


def _mid(a0v, a1v, c0v, c1v, x, wl, wr, b):
    blk = pl.BlockSpec((_BR, D), lambda i: (i, 0))
    cblk = pl.BlockSpec((_BR, 1), lambda i: (i, 0))
    wblk = pl.BlockSpec((D, D), lambda i: (0, 0))
    bblk = pl.BlockSpec((1, D), lambda i: (0, 0))
    return pl.pallas_call(
        _mid_body,
        grid=(N // _BR,),
        in_specs=[blk, blk, cblk, cblk, blk, wblk, wblk, bblk],
        out_specs=blk,
        out_shape=jax.ShapeDtypeStruct((N, D), jnp.float32),
    )(a0v, a1v, c0v, c1v, x, wl, wr, b)


def _fin_body(a0, a1, c0, c1, h_ref, wl, wr, b, o_ref):
    cnt = c0[...] + c1[...]
    inv = 1.0 / jnp.maximum(cnt, 1.0)
    mean = (a0[...] + a1[...]) * inv
    o_ref[...] = _xwt(mean, wl[...]) + b[...] + _xwt(h_ref[...], wr[...])


def _fin(a0v, a1v, c0v, c1v, h, wl, wr, b):
    blk = pl.BlockSpec((_BR, D), lambda i: (i, 0))
    cblk = pl.BlockSpec((_BR, 1), lambda i: (i, 0))
    wblk = pl.BlockSpec((D, D), lambda i: (0, 0))
    bblk = pl.BlockSpec((1, D), lambda i: (0, 0))
    return pl.pallas_call(
        _fin_body,
        grid=(N // _BR,),
        in_specs=[blk, blk, cblk, cblk, blk, wblk, wblk, bblk],
        out_specs=blk,
        out_shape=jax.ShapeDtypeStruct((N, D), jnp.float32),
    )(a0v, a1v, c0v, c1v, h, wl, wr, b)


def kernel(x, edge_index, Wl1, bl1, Wr1, Wl2, bl2, Wr2):
    src = edge_index[0].astype(jnp.int32)
    dst = edge_index[1].astype(jnp.int32)
    padn = EPAD - E
    ar = jnp.arange(padn, dtype=jnp.int32)
    pad_src = (ar * 37) % N
    pad_dst = N + (ar % NPADROWS)
    srcp = jnp.concatenate([src, pad_src])
    dstp = jnp.concatenate([dst, pad_dst])
    srcg = srcp.reshape(NW * NCH, CH)
    dstg = dstp.reshape(NW * NCH, CH)
    dstgc = dstp.reshape(NW * NCHC, CHC)
    zacc = jnp.zeros((NACC, D), jnp.float32)
    ones = jnp.ones((CHC, D), jnp.float32)
    bl1r = bl1.reshape(1, D)
    bl2r = bl2.reshape(1, D)

    cntf = _sc_cnt(dstgc, zacc, ones)
    cntc = cntf[:, :1]
    c0, c1 = cntc[:N], cntc[NACC:NACC + N]

    aggf = _sc_seg(x, srcg, dstg, zacc)
    h = _mid(aggf[:N], aggf[NACC:NACC + N], c0, c1, x, Wl1, Wr1, bl1r)
    aggf2 = _sc_seg(h, srcg, dstg, zacc)
    return _fin(aggf2[:N], aggf2[NACC:NACC + N], c0, c1, h, Wl2, Wr2, bl2r)


# speedup vs baseline: 1.1685x; 1.0191x over previous
import jax
import jax.numpy as jnp
from jax import lax
from jax.experimental import pallas as pl
from jax.experimental.pallas import tpu as pltpu
from jax.experimental.pallas import tpu_sc as plsc

N = 10000
D = 128
E = 320000

NC = 2
NS = 16
NW = NC * NS

EPT = 10240
EPAD = EPT * NW
NPADROWS = 112
NACC = N + NPADROWS
RPT = NACC // NS

CH = 128
NCH = EPT // CH
RB = 2
IB = 16
NIB = NCH // IB

CHC = 128
NCHC = EPT // CHC
IBC = 16
NIBC = NCHC // IBC

_DOT = dict(preferred_element_type=jnp.float32, precision=lax.Precision.HIGHEST)


def _xwt(a, w):
    return lax.dot_general(a, w, (((1,), (1,)), ((), ())), **_DOT)


def _sc_body(table, idxg, zacc, agg_out,
             sd_v, r0, r1, acc_sh, s0, s1):
    c = lax.axis_index("c")
    s = lax.axis_index("s")
    wid = c * NS + s
    lo = s * RPT

    pltpu.sync_copy(zacc.at[pl.ds(lo, RPT)], acc_sh.at[pl.ds(lo, RPT)])
    plsc.subcore_barrier()

    bufs = (r0, r1)
    sems = (s0, s1)

    def outer(jb, carry):
        ib0 = (wid * NIB + jb) * 2 * IB
        pltpu.sync_copy(idxg.at[pl.ds(ib0, 2 * IB)], sd_v)

        h = [None] * IB
        for j in range(RB - 1):
            h[j] = pltpu.async_copy(table.at[sd_v.at[j]], bufs[j % RB],
                                    sems[j % RB])
        for j in range(IB):
            jn = j + RB - 1
            if jn < IB:
                h[jn] = pltpu.async_copy(table.at[sd_v.at[jn]],
                                         bufs[jn % RB], sems[jn % RB])
            h[j].wait()
            pltpu.sync_copy(bufs[j % RB], acc_sh.at[sd_v.at[IB + j]],
                            add=True)
        return carry

    lax.fori_loop(0, NIB, outer, 0)
    plsc.subcore_barrier()

    dlo = c * NACC + lo
    pltpu.sync_copy(acc_sh.at[pl.ds(lo, RPT)], agg_out.at[pl.ds(dlo, RPT)])


_sc_seg = pl.kernel(
    _sc_body,
    out_type=jax.ShapeDtypeStruct((NC * NACC, D), jnp.float32),
    mesh=plsc.VectorSubcoreMesh(core_axis_name="c", subcore_axis_name="s"),
    scratch_types=[
        pltpu.VMEM((2 * IB, CH), jnp.int32),
        pltpu.VMEM((CH, D), jnp.float32),
        pltpu.VMEM((CH, D), jnp.float32),
        pltpu.VMEM_SHARED((NACC, D), jnp.float32),
        pltpu.SemaphoreType.DMA,
        pltpu.SemaphoreType.DMA,
    ],
)


def _cnt_body(dstg, zcnt, ones, cnt_out, dst_v, ones_v, cnt_sh):
    c = lax.axis_index("c")
    s = lax.axis_index("s")
    wid = c * NS + s
    lo = s * RPT

    pltpu.sync_copy(zcnt.at[pl.ds(lo, RPT)], cnt_sh.at[pl.ds(lo, RPT)])
    pltpu.sync_copy(ones, ones_v)
    plsc.subcore_barrier()

    def outer(jb, carry):
        ib0 = wid * NCHC + jb * IBC
        pltpu.sync_copy(dstg.at[pl.ds(ib0, IBC)], dst_v)

        def inner(j, c2):
            pltpu.sync_copy(ones_v, cnt_sh.at[dst_v.at[j]], add=True)
            return c2

        return lax.fori_loop(0, IBC, inner, carry)

    lax.fori_loop(0, NIBC, outer, 0)
    plsc.subcore_barrier()

    dlo = c * NACC + lo
    pltpu.sync_copy(cnt_sh.at[pl.ds(lo, RPT)], cnt_out.at[pl.ds(dlo, RPT)])


_sc_cnt = pl.kernel(
    _cnt_body,
    out_type=jax.ShapeDtypeStruct((NC * NACC, D), jnp.float32),
    mesh=plsc.VectorSubcoreMesh(core_axis_name="c", subcore_axis_name="s"),
    scratch_types=[
        pltpu.VMEM((IBC, CHC), jnp.int32),
        pltpu.VMEM((CHC, D), jnp.float32),
        pltpu.VMEM_SHARED((NACC, D), jnp.float32),
    ],
)


_BR = 1000


def _mid_body(a0, a1, c0, c1, x_ref, wl, wr, b, h_ref):
    cnt = c0[...] + c1[...]
    inv = 1.0 / jnp.maximum(cnt, 1.0)
    mean = (a0[...] + a1[...]) * inv
    h = _xwt(mean, wl[...]) + b[...] + _xwt(x_ref[...], wr[...])
    h_ref[...] = jnp.maximum(h, 0.0)


def _mid(a0v, a1v, c0v, c1v, x, wl, wr, b):
    blk = pl.BlockSpec((_BR, D), lambda i: (i, 0))
    cblk = pl.BlockSpec((_BR, 1), lambda i: (i, 0))
    wblk = pl.BlockSpec((D, D), lambda i: (0, 0))
    bblk = pl.BlockSpec((1, D), lambda i: (0, 0))
    return pl.pallas_call(
        _mid_body,
        grid=(N // _BR,),
        in_specs=[blk, blk, cblk, cblk, blk, wblk, wblk, bblk],
        out_specs=blk,
        out_shape=jax.ShapeDtypeStruct((N, D), jnp.float32),
    )(a0v, a1v, c0v, c1v, x, wl, wr, b)


def _fin_body(a0, a1, c0, c1, h_ref, wl, wr, b, o_ref):
    cnt = c0[...] + c1[...]
    inv = 1.0 / jnp.maximum(cnt, 1.0)
    mean = (a0[...] + a1[...]) * inv
    o_ref[...] = _xwt(mean, wl[...]) + b[...] + _xwt(h_ref[...], wr[...])


def _fin(a0v, a1v, c0v, c1v, h, wl, wr, b):
    blk = pl.BlockSpec((_BR, D), lambda i: (i, 0))
    cblk = pl.BlockSpec((_BR, 1), lambda i: (i, 0))
    wblk = pl.BlockSpec((D, D), lambda i: (0, 0))
    bblk = pl.BlockSpec((1, D), lambda i: (0, 0))
    return pl.pallas_call(
        _fin_body,
        grid=(N // _BR,),
        in_specs=[blk, blk, cblk, cblk, blk, wblk, wblk, bblk],
        out_specs=blk,
        out_shape=jax.ShapeDtypeStruct((N, D), jnp.float32),
    )(a0v, a1v, c0v, c1v, h, wl, wr, b)


def kernel(x, edge_index, Wl1, bl1, Wr1, Wl2, bl2, Wr2):
    src = edge_index[0].astype(jnp.int32)
    dst = edge_index[1].astype(jnp.int32)
    padn = EPAD - E
    ar = jnp.arange(padn, dtype=jnp.int32)
    pad_src = (ar * 37) % N
    pad_dst = N + (ar % NPADROWS)
    srcp = jnp.concatenate([src, pad_src])
    dstp = jnp.concatenate([dst, pad_dst])
    srcg4 = srcp.reshape(NW, NIB, IB, CH)
    dstg4 = dstp.reshape(NW, NIB, IB, CH)
    idxg = jnp.stack([srcg4, dstg4], axis=2).reshape(NW * NIB * 2 * IB, CH)
    dstgc = dstp.reshape(NW * NCHC, CHC)
    zacc = jnp.zeros((NACC, D), jnp.float32)
    ones = jnp.ones((CHC, D), jnp.float32)
    bl1r = bl1.reshape(1, D)
    bl2r = bl2.reshape(1, D)

    cntf = _sc_cnt(dstgc, zacc, ones)
    cntc = cntf[:, :1]
    c0, c1 = cntc[:N], cntc[NACC:NACC + N]

    aggf = _sc_seg(x, idxg, zacc)
    h = _mid(aggf[:N], aggf[NACC:NACC + N], c0, c1, x, Wl1, Wr1, bl1r)
    aggf2 = _sc_seg(h, idxg, zacc)
    return _fin(aggf2[:N], aggf2[NACC:NACC + N], c0, c1, h, Wl2, Wr2, bl2r)


# speedup vs baseline: 1.1765x; 1.0068x over previous
import jax
import jax.numpy as jnp
from jax import lax
from jax.experimental import pallas as pl
from jax.experimental.pallas import tpu as pltpu
from jax.experimental.pallas import tpu_sc as plsc

N = 10000
D = 128
E = 320000

NC = 2
NS = 16
NW = NC * NS

EPT = 10240
EPAD = EPT * NW
NPADROWS = 112
NACC = N + NPADROWS
RPT = NACC // NS

CH = 128
NCH = EPT // CH
RB = 2
IB = 16
NIB = NCH // IB

_DOT = dict(preferred_element_type=jnp.float32, precision=lax.Precision.HIGHEST)


def _xwt(a, w):
    return lax.dot_general(a, w, (((1,), (1,)), ((), ())), **_DOT)


def _sc_body(table, idxg, zacc, agg_out,
             sd_v, r0, r1, acc_sh, s0, s1):
    c = lax.axis_index("c")
    s = lax.axis_index("s")
    wid = c * NS + s
    lo = s * RPT

    pltpu.sync_copy(zacc.at[pl.ds(lo, RPT)], acc_sh.at[pl.ds(lo, RPT)])
    plsc.subcore_barrier()

    bufs = (r0, r1)
    sems = (s0, s1)

    def outer(jb, carry):
        ib0 = (wid * NIB + jb) * 2 * IB
        pltpu.sync_copy(idxg.at[pl.ds(ib0, 2 * IB)], sd_v)

        h = [None] * IB
        for j in range(RB - 1):
            h[j] = pltpu.async_copy(table.at[sd_v.at[j]], bufs[j % RB],
                                    sems[j % RB])
        for j in range(IB):
            jn = j + RB - 1
            if jn < IB:
                h[jn] = pltpu.async_copy(table.at[sd_v.at[jn]],
                                         bufs[jn % RB], sems[jn % RB])
            h[j].wait()
            pltpu.sync_copy(bufs[j % RB], acc_sh.at[sd_v.at[IB + j]],
                            add=True)
        return carry

    lax.fori_loop(0, NIB, outer, 0)
    plsc.subcore_barrier()

    pltpu.sync_copy(acc_sh.at[pl.ds(lo, RPT)], agg_out.at[pl.ds(lo, RPT), c])


_sc_seg = pl.kernel(
    _sc_body,
    out_type=jax.ShapeDtypeStruct((NACC, NC, D), jnp.float32),
    mesh=plsc.VectorSubcoreMesh(core_axis_name="c", subcore_axis_name="s"),
    scratch_types=[
        pltpu.VMEM((2 * IB, CH), jnp.int32),
        pltpu.VMEM((CH, D), jnp.float32),
        pltpu.VMEM((CH, D), jnp.float32),
        pltpu.VMEM_SHARED((NACC, D), jnp.float32),
        pltpu.SemaphoreType.DMA,
        pltpu.SemaphoreType.DMA,
    ],
)


def _cnt_body(idxg, zcnt, ones, cnt_out, dst_v, ones_v, cnt_sh):
    c = lax.axis_index("c")
    s = lax.axis_index("s")
    wid = c * NS + s
    lo = s * RPT

    pltpu.sync_copy(zcnt.at[pl.ds(lo, RPT)], cnt_sh.at[pl.ds(lo, RPT)])
    pltpu.sync_copy(ones, ones_v)
    plsc.subcore_barrier()

    def outer(jb, carry):
        ib0 = (wid * NIB + jb) * 2 * IB + IB
        pltpu.sync_copy(idxg.at[pl.ds(ib0, IB)], dst_v)

        def inner(j, c2):
            pltpu.sync_copy(ones_v, cnt_sh.at[dst_v.at[j]], add=True)
            return c2

        return lax.fori_loop(0, IB, inner, carry)

    lax.fori_loop(0, NIB, outer, 0)
    plsc.subcore_barrier()

    dlo = c * NACC + lo
    pltpu.sync_copy(cnt_sh.at[pl.ds(lo, RPT)], cnt_out.at[pl.ds(dlo, RPT)])


_sc_cnt = pl.kernel(
    _cnt_body,
    out_type=jax.ShapeDtypeStruct((NC * NACC, D), jnp.float32),
    mesh=plsc.VectorSubcoreMesh(core_axis_name="c", subcore_axis_name="s"),
    scratch_types=[
        pltpu.VMEM((IB, CH), jnp.int32),
        pltpu.VMEM((CH, D), jnp.float32),
        pltpu.VMEM_SHARED((NACC, D), jnp.float32),
    ],
)


_BR = 1000


def _mid_body(a, c0, c1, x_ref, wl, wr, b, h_ref):
    cnt = c0[...] + c1[...]
    inv = 1.0 / jnp.maximum(cnt, 1.0)
    av = a[...]
    mean = (av[:, 0, :] + av[:, 1, :]) * inv
    h = _xwt(mean, wl[...]) + b[...] + _xwt(x_ref[...], wr[...])
    h_ref[...] = jnp.maximum(h, 0.0)


def _mid(av, c0v, c1v, x, wl, wr, b):
    blk = pl.BlockSpec((_BR, D), lambda i: (i, 0))
    ablk = pl.BlockSpec((_BR, NC, D), lambda i: (i, 0, 0))
    cblk = pl.BlockSpec((_BR, 1), lambda i: (i, 0))
    wblk = pl.BlockSpec((D, D), lambda i: (0, 0))
    bblk = pl.BlockSpec((1, D), lambda i: (0, 0))
    return pl.pallas_call(
        _mid_body,
        grid=(N // _BR,),
        in_specs=[ablk, cblk, cblk, blk, wblk, wblk, bblk],
        out_specs=blk,
        out_shape=jax.ShapeDtypeStruct((N, D), jnp.float32),
    )(av, c0v, c1v, x, wl, wr, b)


def _fin_body(a, c0, c1, h_ref, wl, wr, b, o_ref):
    cnt = c0[...] + c1[...]
    inv = 1.0 / jnp.maximum(cnt, 1.0)
    av = a[...]
    mean = (av[:, 0, :] + av[:, 1, :]) * inv
    o_ref[...] = _xwt(mean, wl[...]) + b[...] + _xwt(h_ref[...], wr[...])


def _fin(av, c0v, c1v, h, wl, wr, b):
    blk = pl.BlockSpec((_BR, D), lambda i: (i, 0))
    ablk = pl.BlockSpec((_BR, NC, D), lambda i: (i, 0, 0))
    cblk = pl.BlockSpec((_BR, 1), lambda i: (i, 0))
    wblk = pl.BlockSpec((D, D), lambda i: (0, 0))
    bblk = pl.BlockSpec((1, D), lambda i: (0, 0))
    return pl.pallas_call(
        _fin_body,
        grid=(N // _BR,),
        in_specs=[ablk, cblk, cblk, blk, wblk, wblk, bblk],
        out_specs=blk,
        out_shape=jax.ShapeDtypeStruct((N, D), jnp.float32),
    )(av, c0v, c1v, h, wl, wr, b)


def kernel(x, edge_index, Wl1, bl1, Wr1, Wl2, bl2, Wr2):
    src = edge_index[0].astype(jnp.int32)
    dst = edge_index[1].astype(jnp.int32)
    padn = EPAD - E
    ar = jnp.arange(padn, dtype=jnp.int32)
    pad_src = (ar * 37) % N
    pad_dst = N + (ar % NPADROWS)
    srcp = jnp.concatenate([src, pad_src])
    dstp = jnp.concatenate([dst, pad_dst])
    srcg4 = srcp.reshape(NW, NIB, IB, CH)
    dstg4 = dstp.reshape(NW, NIB, IB, CH)
    idxg = jnp.stack([srcg4, dstg4], axis=2).reshape(NW * NIB * 2 * IB, CH)
    zacc = jnp.zeros((NACC, D), jnp.float32)
    ones = jnp.ones((CH, D), jnp.float32)
    bl1r = bl1.reshape(1, D)
    bl2r = bl2.reshape(1, D)

    cntf = _sc_cnt(idxg, zacc, ones)
    cntc = cntf[:, :1]
    c0, c1 = cntc[:N], cntc[NACC:NACC + N]

    aggf = _sc_seg(x, idxg, zacc)
    h = _mid(aggf, c0, c1, x, Wl1, Wr1, bl1r)
    aggf2 = _sc_seg(h, idxg, zacc)
    return _fin(aggf2, c0, c1, h, Wl2, Wr2, bl2r)
